# Initial kernel scaffold; baseline (speedup 1.0000x reference)
import jax
import jax.numpy as jnp
from jax.experimental import pallas as pl


def kernel(x, adj, W, b):
    raise NotImplementedError("write your pallas kernel here")


# speedup vs baseline: 1.8627x; 1.8627x over previous
import functools

import jax
import jax.numpy as jnp
from jax.experimental import pallas as pl

_B, _N, _C = 8, 256, 128
_T_THRESH, _K_SKIP = 10, 2
_TAUS = (0.25, 0.5, 0.75)
_WTS = (0.25, 0.5, 0.25)
_TILE = 8


def _dilate_kernel(adj_ref, out_ref):
    a = adj_ref[0]
    row = jax.lax.broadcasted_iota(jnp.int32, (_N, _N), 0)
    col = jax.lax.broadcasted_iota(jnp.int32, (_N, _N), 1)
    eye = row == col
    adj_ = jnp.where(eye, 1.0, a)
    nmf = jnp.where(eye, 0.0, (adj_ > 0).astype(jnp.float32))
    num = jnp.sum(nmf, axis=1, keepdims=True)
    skip = jnp.where(num > _T_THRESH,
                     jnp.floor((num + (_K_SKIP - 1)) / _K_SKIP), 1.0)
    ltri = (row < col).astype(jnp.float32)
    rank = jax.lax.dot_general(nmf, ltri, (((1,), (0,)), ((), ())),
                               preferred_element_type=jnp.float32)
    q = jnp.floor(rank / skip)
    modr = rank - q * skip
    rm = jnp.where((nmf > 0) & (skip > 1.0) & (modr == skip - 1.0), 1.0, 0.0)
    rm_sym = jnp.maximum(rm, rm.T)
    out_ref[0] = jnp.where(rm_sym > 0, 0.0, adj_)


def _quant_kernel(xt_ref, xtile_ref, adjd_ref, w_ref, b_ref, out_ref):
    xfull = xt_ref[0]
    mrow = adjd_ref[0]
    m = jnp.sum(mrow, axis=2, keepdims=True)

    inf = jnp.float32(jnp.inf)
    vals = jnp.where(mrow > 0, xfull[None, :, :], inf)

    idx = jax.lax.broadcasted_iota(jnp.int32, (1, 1, _N), 2)
    logn = _N.bit_length() - 1
    for k in range(1, logn + 1):
        asc = (idx & (1 << k)) == 0
        for j in range(k - 1, -1, -1):
            d = 1 << j
            bit_d = (idx & d) != 0
            p = jnp.where(bit_d, jnp.roll(vals, d, axis=2),
                          jnp.roll(vals, -d, axis=2))
            mn = jnp.minimum(vals, p)
            mx = jnp.maximum(vals, p)
            take_min = asc != bit_d
            vals = jnp.where(take_min, mn, mx)

    ji = jax.lax.broadcasted_iota(jnp.int32, (1, 1, _N), 2)
    g = jnp.zeros((_TILE, 1, _N), dtype=jnp.float32)
    for tau, wt in zip(_TAUS, _WTS):
        pos = tau * (m - 1.0)
        klo = jnp.floor(pos)
        frac = pos - klo
        khi = jnp.minimum(klo + 1.0, m - 1.0)
        g = g + wt * ((1.0 - frac) * (ji == klo.astype(jnp.int32)).astype(jnp.float32)
                      + frac * (ji == khi.astype(jnp.int32)).astype(jnp.float32))

    cleaned = jnp.where(ji < m.astype(jnp.int32), vals, 0.0)
    agg = jnp.sum(g * cleaned, axis=2)

    pre = xtile_ref[0] + agg
    out = jax.lax.dot_general(pre, w_ref[...], (((1,), (1,)), ((), ())),
                              preferred_element_type=jnp.float32)
    out_ref[0] = out + b_ref[...]


@functools.partial(jax.jit, static_argnames=())
def kernel(x, adj, W, b):
    adjd = pl.pallas_call(
        _dilate_kernel,
        grid=(_B,),
        in_specs=[pl.BlockSpec((1, _N, _N), lambda i: (i, 0, 0))],
        out_specs=pl.BlockSpec((1, _N, _N), lambda i: (i, 0, 0)),
        out_shape=jax.ShapeDtypeStruct((_B, _N, _N), jnp.float32),
    )(adj)

    b2 = b.reshape(1, _C)
    xT = jnp.swapaxes(x, 1, 2)
    adjd4 = adjd.reshape(_B, _N, 1, _N)
    nt = _N // _TILE
    out = pl.pallas_call(
        _quant_kernel,
        grid=(_B, nt),
        in_specs=[
            pl.BlockSpec((1, _C, _N), lambda i, t: (i, 0, 0)),
            pl.BlockSpec((1, _TILE, _C), lambda i, t: (i, t, 0)),
            pl.BlockSpec((1, _TILE, 1, _N), lambda i, t: (i, t, 0, 0)),
            pl.BlockSpec((_C, _C), lambda i, t: (0, 0)),
            pl.BlockSpec((1, _C), lambda i, t: (0, 0)),
        ],
        out_specs=pl.BlockSpec((1, _TILE, _C), lambda i, t: (i, t, 0)),
        out_shape=jax.ShapeDtypeStruct((_B, _N, _C), jnp.float32),
    )(xT, x, adjd4, W, b2)
    return out
